# baseline (speedup 1.0000x reference)
import functools

import jax
import jax.numpy as jnp
from jax import lax
from jax.experimental import pallas as pl
from jax.experimental.pallas import tpu as pltpu
from jax.experimental.pallas import tpu_sc as plsc

NC = 2
NS = 16
L = 16
BLK = 80
HW = 80
AW = 144
H = 8
C = 16
NEG_SLOPE = 0.2


def _gather16(v, idx):
    return lax.gather(
        v,
        idx[:, None],
        lax.GatherDimensionNumbers(
            offset_dims=(), collapsed_slice_dims=(0,), start_index_map=(0,)
        ),
        (1,),
        mode=lax.GatherScatterMode.PROMISE_IN_BOUNDS,
    )


def _sc_edge_kernel(np_rows, nbw0, nbw1):
    rows_per_tile = np_rows // NS
    mesh = plsc.VectorSubcoreMesh(core_axis_name="c", subcore_axis_name="s",
                                  num_cores=NC, num_subcores=NS)

    @functools.partial(
        pl.kernel,
        out_type=jax.ShapeDtypeStruct((NC, np_rows, AW), jnp.float32),
        mesh=mesh,
        compiler_params=pltpu.CompilerParams(use_tc_tiling_on_sc=False,
                                             needs_layout_passes=False),
        scratch_types=[
            pltpu.VMEM_SHARED((np_rows, AW), jnp.float32),
            pltpu.VMEM((4, 2, BLK), jnp.int32),
            pltpu.VMEM((2, BLK, HW), jnp.int32),
            pltpu.VMEM((2, BLK, L), jnp.float32),
            pltpu.VMEM((2, BLK, AW), jnp.float32),
            pltpu.SemaphoreType.DMA,
            pltpu.SemaphoreType.DMA,
            pltpu.SemaphoreType.DMA,
            pltpu.SemaphoreType.DMA,
            pltpu.SemaphoreType.DMA,
        ],
    )
    def k(hx_hbm, acat_hbm, src_hbm, dst_hbm, acc_out,
          acc_sp, idx4, hb2, abuf_d2, hrow2,
          sg0, sg1, ss0, ss1, si):
        c = lax.axis_index("c")
        s = lax.axis_index("s")
        r0 = s * rows_per_tile
        base_blk = jnp.where(c == 0, s * nbw0, NS * nbw0 + s * nbw1)
        my_nb = jnp.where(c == 0, nbw0, nbw1)
        sg = (sg0, sg1)
        ss = (ss0, ss1)

        zv = jnp.zeros((L,), jnp.float32)
        hrow0 = hrow2.at[0]

        def zero_body(i, _):
            for cc in range(AW // L):
                hrow0[i, pl.ds(cc * L, L)] = zv
            return 0

        lax.fori_loop(0, BLK, zero_body, 0)

        off = 0
        rem = rows_per_tile
        while rem > 0:
            sz = min(BLK, rem)
            pltpu.sync_copy(hrow0.at[pl.ds(0, sz)],
                            acc_sp.at[pl.ds(r0 + off, sz)])
            off += sz
            rem -= sz

        plsc.subcore_barrier()

        def issue_idx(b, slot):
            o = (base_blk + b) * BLK
            pltpu.async_copy(src_hbm.at[pl.ds(o, BLK)], idx4.at[slot, 0], si)
            pltpu.async_copy(dst_hbm.at[pl.ds(o, BLK)], idx4.at[slot, 1], si)

        def wait_idx(slot):
            pltpu.make_async_copy(src_hbm.at[pl.ds(0, BLK)],
                                  idx4.at[slot, 0], si).wait()
            pltpu.make_async_copy(src_hbm.at[pl.ds(0, BLK)],
                                  idx4.at[slot, 1], si).wait()

        def issue_gathers(kp, slot):
            pltpu.async_copy(hx_hbm.at[idx4.at[slot, 0]], hb2.at[kp], sg[kp])
            pltpu.async_copy(acat_hbm.at[idx4.at[slot, 1]],
                             abuf_d2.at[kp], sg[kp])

        def drain_gathers(kp):
            pltpu.make_async_copy(hx_hbm.at[idx4.at[0, 0]],
                                  hb2.at[kp], sg[kp]).wait()
            pltpu.make_async_copy(acat_hbm.at[idx4.at[0, 0]],
                                  abuf_d2.at[kp], sg[kp]).wait()

        def issue_scatter(kp, slot):
            pltpu.async_copy(hrow2.at[kp], acc_sp.at[idx4.at[slot, 1]],
                             ss[kp], add=True)

        def drain_scatter(kp):
            pltpu.make_async_copy(hrow2.at[kp],
                                  acc_sp.at[idx4.at[0, 1]], ss[kp]).wait()

        pltpu.sync_copy(src_hbm.at[pl.ds(base_blk * BLK, BLK)],
                        idx4.at[0, 0])
        pltpu.sync_copy(dst_hbm.at[pl.ds(base_blk * BLK, BLK)],
                        idx4.at[0, 1])
        issue_idx(1, 1)
        issue_gathers(0, 0)

        def process(b, k4):
            kp = k4 % 2

            @pl.when(b >= 2)
            def _():
                drain_scatter(kp)

            @pl.when(b + 1 < my_nb)
            def _():
                wait_idx((k4 + 1) % 4)

            @pl.when(b + 2 < my_nb)
            def _():
                issue_idx(b + 2, (k4 + 2) % 4)

            @pl.when(b + 1 < my_nb)
            def _():
                issue_gathers(1 - kp, (k4 + 1) % 4)

            drain_gathers(kp)

            hb = hb2.at[kp]
            abuf_d = abuf_d2.at[kp]
            hrow = hrow2.at[kp]

            @plsc.parallel_loop(0, BLK, step=1, unroll=2)
            def edge_body(j):
                v_s = plsc.bitcast(hb[j, pl.ds(HW - 24, L)], jnp.float32)
                v_d = abuf_d[j, :]
                t = v_s + v_d
                w = jnp.exp(jnp.maximum(t, t * NEG_SLOPE))
                hrow[j, pl.ds(H * C, L)] = w
                for hh in range(4):
                    v = hb[j, pl.ds(hh * L, L)]
                    va = plsc.bitcast(v << 16, jnp.float32)
                    vb = plsc.bitcast(v & jnp.int32(-65536), jnp.float32)
                    wa = _gather16(w, jnp.full((L,), 8 + hh, jnp.int32))
                    wb = _gather16(w, jnp.full((L,), 12 + hh, jnp.int32))
                    hrow[j, pl.ds(hh * L, L)] = va * wa
                    hrow[j, pl.ds(64 + hh * L, L)] = vb * wb

            issue_scatter(kp, k4)

        def block_body(i, _):
            b = i * 4
            for k4 in range(4):
                process(b + k4, k4)
            return 0

        lax.fori_loop(0, my_nb // 4, block_body, 0)

        drain_scatter(0)
        drain_scatter(1)

        plsc.subcore_barrier()

        pltpu.sync_copy(acc_sp.at[pl.ds(r0, rows_per_tile)],
                        acc_out.at[c, pl.ds(r0, rows_per_tile)])

    return k


def _bf16_bits(x):
    b = lax.bitcast_convert_type(x, jnp.int32)
    return (b + jnp.int32(0x7FFF) + ((b >> 16) & 1)) >> 16


def _proj_body(x_ref, w_ref, ab_ref, hx_ref, acat_ref):
    n = x_ref.shape[0]
    pad = hx_ref.shape[0] - n
    half = H * C // 2
    hv = jnp.dot(x_ref[...], w_ref[...], preferred_element_type=jnp.float32)
    acat = jnp.dot(hv, ab_ref[...], preferred_element_type=jnp.float32)
    lo = _bf16_bits(hv[:, :half]) & jnp.int32(0xFFFF)
    hi = _bf16_bits(hv[:, half:]) << 16
    hx_ref[pl.ds(0, n), pl.ds(0, half)] = lo | hi
    hx_ref[pl.ds(0, n), pl.ds(half, H)] = lax.bitcast_convert_type(
        acat[:, :H], jnp.int32)
    hx_ref[pl.ds(0, n), pl.ds(half + H, HW - half - H)] = jnp.zeros(
        (n, HW - half - H), jnp.int32)
    hx_ref[pl.ds(n, pad)] = jnp.zeros((pad, HW), jnp.int32)
    acat_ref[pl.ds(0, n)] = acat
    acat_ref[pl.ds(n, pad)] = jnp.zeros((pad, 2 * H), jnp.float32)


def _combine_body(a0_ref, a1_ref, p8_ref, b_ref, o_ref):
    n = o_ref.shape[0]
    asum = a0_ref[pl.ds(0, n)] + a1_ref[pl.ds(0, n)]
    den = jnp.dot(asum[:, H * C + H:], p8_ref[...],
                  preferred_element_type=jnp.float32) + 1e-16
    o_ref[...] = asum[:, :H * C] / den + b_ref[...]


def kernel(x, edge_index, W, att_src, att_dst, bias):
    n, f_in = x.shape
    e = edge_index.shape[1]
    f_out = W.shape[1]

    np_rows = -(-(n + 1) // (NS * 8)) * (NS * 8)

    eye = jnp.eye(H, dtype=jnp.float32)
    a_s = (att_src.reshape(H, C)[:, :, None] * eye[:, None, :]).reshape(H * C, H)
    a_d = (att_dst.reshape(H, C)[:, :, None] * eye[:, None, :]).reshape(H * C, H)
    ab = jnp.concatenate([a_s, a_d], axis=1)

    hx, acat = pl.pallas_call(
        _proj_body,
        out_shape=[
            jax.ShapeDtypeStruct((np_rows, HW), jnp.int32),
            jax.ShapeDtypeStruct((np_rows, 2 * H), jnp.float32),
        ],
    )(x, W, ab)

    total = e + n
    nbw = -(-total // (NC * NS * BLK))
    nbw = -(-nbw // 4) * 4
    nbw0 = (nbw * 2 * 7) // 12
    nbw0 = -(-nbw0 // 4) * 4
    nbw1 = 2 * nbw - nbw0
    e_pad = (nbw0 + nbw1) * NS * BLK
    loops = jnp.arange(n, dtype=jnp.int32)
    fill = jnp.full((e_pad - total,), n, jnp.int32)
    src_flat = jnp.concatenate([edge_index[0], loops, fill])
    dst_flat = jnp.concatenate([edge_index[1], loops, fill])

    acc_parts = _sc_edge_kernel(np_rows, nbw0, nbw1)(
        hx, acat, src_flat, dst_flat)

    p8 = jnp.repeat(jnp.eye(H, dtype=jnp.float32), C, axis=1)
    out = pl.pallas_call(
        _combine_body,
        out_shape=jax.ShapeDtypeStruct((n, f_out), jnp.float32),
    )(acc_parts[0], acc_parts[1], p8, bias.reshape(1, f_out))

    return out


# speedup vs baseline: 1.5080x; 1.5080x over previous
import functools

import jax
import jax.numpy as jnp
from jax import lax
from jax.experimental import pallas as pl
from jax.experimental.pallas import tpu as pltpu
from jax.experimental.pallas import tpu_sc as plsc

NC = 2
NS = 16
L = 16
BLK = 56
HW = 80
AW = 144
H = 8
C = 16
NEG_SLOPE = 0.2


def _gather16(v, idx):
    return lax.gather(
        v,
        idx[:, None],
        lax.GatherDimensionNumbers(
            offset_dims=(), collapsed_slice_dims=(0,), start_index_map=(0,)
        ),
        (1,),
        mode=lax.GatherScatterMode.PROMISE_IN_BOUNDS,
    )


def _sc_edge_kernel(np_rows, nbw0, nbw1):
    rows_per_tile = np_rows // NS
    mesh = plsc.VectorSubcoreMesh(core_axis_name="c", subcore_axis_name="s",
                                  num_cores=NC, num_subcores=NS)

    @functools.partial(
        pl.kernel,
        out_type=jax.ShapeDtypeStruct((NC, np_rows, AW), jnp.float32),
        mesh=mesh,
        compiler_params=pltpu.CompilerParams(use_tc_tiling_on_sc=False,
                                             needs_layout_passes=False),
        scratch_types=[
            pltpu.VMEM_SHARED((np_rows, AW), jnp.float32),
            pltpu.VMEM((6, 2, BLK), jnp.int32),
            pltpu.VMEM((2, BLK, HW), jnp.int32),
            pltpu.VMEM((3, BLK, L), jnp.float32),
            pltpu.VMEM((3, BLK, AW), jnp.float32),
            pltpu.SemaphoreType.DMA,
            pltpu.SemaphoreType.DMA,
            pltpu.SemaphoreType.DMA,
            pltpu.SemaphoreType.DMA,
            pltpu.SemaphoreType.DMA,
            pltpu.SemaphoreType.DMA,
            pltpu.SemaphoreType.DMA,
        ],
    )
    def k(hx_hbm, acat_hbm, src_hbm, dst_hbm, acc_out,
          acc_sp, idx6, hb2, abuf_d3, hrow3,
          sg0, sg1, sg2, ss0, ss1, ss2, si):
        c = lax.axis_index("c")
        s = lax.axis_index("s")
        r0 = s * rows_per_tile
        base_blk = jnp.where(c == 0, s * nbw0, NS * nbw0 + s * nbw1)
        my_nb = jnp.where(c == 0, nbw0, nbw1)
        sg = (sg0, sg1, sg2)
        ss = (ss0, ss1, ss2)

        zv = jnp.zeros((L,), jnp.float32)
        hrow0 = hrow3.at[0]

        def zero_body(i, _):
            for cc in range(AW // L):
                hrow0[i, pl.ds(cc * L, L)] = zv
            return 0

        lax.fori_loop(0, BLK, zero_body, 0)

        off = 0
        rem = rows_per_tile
        while rem > 0:
            sz = min(BLK, rem)
            pltpu.sync_copy(hrow0.at[pl.ds(0, sz)],
                            acc_sp.at[pl.ds(r0 + off, sz)])
            off += sz
            rem -= sz

        plsc.subcore_barrier()

        def issue_idx(b, slot):
            o = (base_blk + b) * BLK
            pltpu.async_copy(src_hbm.at[pl.ds(o, BLK)], idx6.at[slot, 0], si)
            pltpu.async_copy(dst_hbm.at[pl.ds(o, BLK)], idx6.at[slot, 1], si)

        def wait_idx(slot):
            pltpu.make_async_copy(src_hbm.at[pl.ds(0, BLK)],
                                  idx6.at[slot, 0], si).wait()
            pltpu.make_async_copy(src_hbm.at[pl.ds(0, BLK)],
                                  idx6.at[slot, 1], si).wait()

        def issue_gathers(q, kp, slot):
            pltpu.async_copy(hx_hbm.at[idx6.at[slot, 0]], hb2.at[kp], sg[q])
            pltpu.async_copy(acat_hbm.at[idx6.at[slot, 1]],
                             abuf_d3.at[q], sg[q])

        def drain_gathers(q, kp):
            pltpu.make_async_copy(hx_hbm.at[idx6.at[0, 0]],
                                  hb2.at[kp], sg[q]).wait()
            pltpu.make_async_copy(acat_hbm.at[idx6.at[0, 0]],
                                  abuf_d3.at[q], sg[q]).wait()

        def issue_scatter(q, slot):
            pltpu.async_copy(hrow3.at[q], acc_sp.at[idx6.at[slot, 1]],
                             ss[q], add=True)

        def drain_scatter(q):
            pltpu.make_async_copy(hrow3.at[q],
                                  acc_sp.at[idx6.at[0, 1]], ss[q]).wait()

        pltpu.sync_copy(src_hbm.at[pl.ds(base_blk * BLK, BLK)],
                        idx6.at[0, 0])
        pltpu.sync_copy(dst_hbm.at[pl.ds(base_blk * BLK, BLK)],
                        idx6.at[0, 1])
        issue_idx(1, 1)
        issue_gathers(0, 0, 0)

        def process(b, k6):
            q = k6 % 3
            qn = (q + 1) % 3
            kp = k6 % 2

            @pl.when(b >= 2)
            def _():
                drain_scatter(qn)

            @pl.when(b + 1 < my_nb)
            def _():
                wait_idx((k6 + 1) % 6)

            @pl.when(b + 2 < my_nb)
            def _():
                issue_idx(b + 2, (k6 + 2) % 6)

            @pl.when(b + 1 < my_nb)
            def _():
                issue_gathers(qn, 1 - kp, (k6 + 1) % 6)

            drain_gathers(q, kp)

            hb = hb2.at[kp]
            abuf_d = abuf_d3.at[q]
            hrow = hrow3.at[q]

            @plsc.parallel_loop(0, BLK, step=1, unroll=2)
            def edge_body(j):
                v_s = plsc.bitcast(hb[j, pl.ds(HW - 24, L)], jnp.float32)
                v_d = abuf_d[j, :]
                t = v_s + v_d
                w = jnp.exp(jnp.maximum(t, t * NEG_SLOPE))
                hrow[j, pl.ds(H * C, L)] = w
                for hh in range(4):
                    v = hb[j, pl.ds(hh * L, L)]
                    va = plsc.bitcast(v << 16, jnp.float32)
                    vb = plsc.bitcast(v & jnp.int32(-65536), jnp.float32)
                    wa = _gather16(w, jnp.full((L,), 8 + hh, jnp.int32))
                    wb = _gather16(w, jnp.full((L,), 12 + hh, jnp.int32))
                    hrow[j, pl.ds(hh * L, L)] = va * wa
                    hrow[j, pl.ds(64 + hh * L, L)] = vb * wb

            issue_scatter(q, k6)

        def block_body(i, _):
            b = i * 6
            for k6 in range(6):
                process(b + k6, k6)
            return 0

        lax.fori_loop(0, my_nb // 6, block_body, 0)

        drain_scatter(1)
        drain_scatter(2)

        plsc.subcore_barrier()

        pltpu.sync_copy(acc_sp.at[pl.ds(r0, rows_per_tile)],
                        acc_out.at[c, pl.ds(r0, rows_per_tile)])

    return k


def _bf16_bits(x):
    b = lax.bitcast_convert_type(x, jnp.int32)
    return (b + jnp.int32(0x7FFF) + ((b >> 16) & 1)) >> 16


def _proj_body(x_ref, w_ref, ab_ref, hx_ref, acat_ref):
    n = x_ref.shape[0]
    pad = hx_ref.shape[0] - n
    half = H * C // 2
    hv = jnp.dot(x_ref[...], w_ref[...], preferred_element_type=jnp.float32)
    acat = jnp.dot(hv, ab_ref[...], preferred_element_type=jnp.float32)
    lo = _bf16_bits(hv[:, :half]) & jnp.int32(0xFFFF)
    hi = _bf16_bits(hv[:, half:]) << 16
    hx_ref[pl.ds(0, n), pl.ds(0, half)] = lo | hi
    hx_ref[pl.ds(0, n), pl.ds(half, H)] = lax.bitcast_convert_type(
        acat[:, :H], jnp.int32)
    hx_ref[pl.ds(0, n), pl.ds(half + H, HW - half - H)] = jnp.zeros(
        (n, HW - half - H), jnp.int32)
    hx_ref[pl.ds(n, pad)] = jnp.zeros((pad, HW), jnp.int32)
    acat_ref[pl.ds(0, n)] = acat
    acat_ref[pl.ds(n, pad)] = jnp.zeros((pad, 2 * H), jnp.float32)


def _combine_body(a0_ref, a1_ref, p8_ref, b_ref, o_ref):
    n = o_ref.shape[0]
    asum = a0_ref[pl.ds(0, n)] + a1_ref[pl.ds(0, n)]
    den = jnp.dot(asum[:, H * C + H:], p8_ref[...],
                  preferred_element_type=jnp.float32) + 1e-16
    o_ref[...] = asum[:, :H * C] / den + b_ref[...]


def kernel(x, edge_index, W, att_src, att_dst, bias):
    n, f_in = x.shape
    e = edge_index.shape[1]
    f_out = W.shape[1]

    np_rows = -(-(n + 1) // (NS * 8)) * (NS * 8)

    eye = jnp.eye(H, dtype=jnp.float32)
    a_s = (att_src.reshape(H, C)[:, :, None] * eye[:, None, :]).reshape(H * C, H)
    a_d = (att_dst.reshape(H, C)[:, :, None] * eye[:, None, :]).reshape(H * C, H)
    ab = jnp.concatenate([a_s, a_d], axis=1)

    hx, acat = pl.pallas_call(
        _proj_body,
        out_shape=[
            jax.ShapeDtypeStruct((np_rows, HW), jnp.int32),
            jax.ShapeDtypeStruct((np_rows, 2 * H), jnp.float32),
        ],
    )(x, W, ab)

    total = e + n
    nbw = -(-total // (NC * NS * BLK))
    nbw = -(-nbw // 6) * 6
    nbw0 = (nbw * 2 * 7) // 12
    nbw0 = -(-nbw0 // 6) * 6
    nbw1 = 2 * nbw - nbw0
    e_pad = (nbw0 + nbw1) * NS * BLK
    loops = jnp.arange(n, dtype=jnp.int32)
    fill = jnp.full((e_pad - total,), n, jnp.int32)
    src_flat = jnp.concatenate([edge_index[0], loops, fill])
    dst_flat = jnp.concatenate([edge_index[1], loops, fill])

    acc_parts = _sc_edge_kernel(np_rows, nbw0, nbw1)(
        hx, acat, src_flat, dst_flat)

    p8 = jnp.repeat(jnp.eye(H, dtype=jnp.float32), C, axis=1)
    out = pl.pallas_call(
        _combine_body,
        out_shape=jax.ShapeDtypeStruct((n, f_out), jnp.float32),
    )(acc_parts[0], acc_parts[1], p8, bias.reshape(1, f_out))

    return out


# speedup vs baseline: 1.5134x; 1.0036x over previous
import functools

import jax
import jax.numpy as jnp
from jax import lax
from jax.experimental import pallas as pl
from jax.experimental.pallas import tpu as pltpu
from jax.experimental.pallas import tpu_sc as plsc

NC = 2
NS = 16
L = 16
BLK = 56
HW = 80
H = 8
C = 16
NEG_SLOPE = 0.2


def _gather16(v, idx):
    return lax.gather(
        v,
        idx[:, None],
        lax.GatherDimensionNumbers(
            offset_dims=(), collapsed_slice_dims=(0,), start_index_map=(0,)
        ),
        (1,),
        mode=lax.GatherScatterMode.PROMISE_IN_BOUNDS,
    )


def _sc_edge_kernel(np_rows, nbw0, nbw1):
    rows_per_tile = np_rows // NS
    mesh = plsc.VectorSubcoreMesh(core_axis_name="c", subcore_axis_name="s",
                                  num_cores=NC, num_subcores=NS)

    @functools.partial(
        pl.kernel,
        out_type=[
            jax.ShapeDtypeStruct((NC, np_rows, H * C), jnp.float32),
            jax.ShapeDtypeStruct((NC, np_rows, L), jnp.float32),
        ],
        mesh=mesh,
        compiler_params=pltpu.CompilerParams(use_tc_tiling_on_sc=False,
                                             needs_layout_passes=False),
        scratch_types=[
            pltpu.VMEM_SHARED((np_rows, H * C), jnp.float32),
            pltpu.VMEM_SHARED((np_rows, L), jnp.float32),
            pltpu.VMEM((6, 2, BLK), jnp.int32),
            pltpu.VMEM((2, BLK, HW), jnp.int32),
            pltpu.VMEM((3, BLK, L), jnp.float32),
            pltpu.VMEM((3, BLK, H * C), jnp.float32),
            pltpu.VMEM((3, BLK, L), jnp.float32),
            pltpu.SemaphoreType.DMA,
            pltpu.SemaphoreType.DMA,
            pltpu.SemaphoreType.DMA,
            pltpu.SemaphoreType.DMA,
            pltpu.SemaphoreType.DMA,
            pltpu.SemaphoreType.DMA,
            pltpu.SemaphoreType.DMA,
        ],
    )
    def k(hx_hbm, acat_hbm, src_hbm, dst_hbm, num_out, den_out,
          num_sp, den_sp, idx6, hb2, abuf_d3, hrow3, wbuf3,
          sg0, sg1, sg2, ss0, ss1, ss2, si):
        c = lax.axis_index("c")
        s = lax.axis_index("s")
        r0 = s * rows_per_tile
        base_blk = jnp.where(c == 0, s * nbw0, NS * nbw0 + s * nbw1)
        my_nb = jnp.where(c == 0, nbw0, nbw1)
        sg = (sg0, sg1, sg2)
        ss = (ss0, ss1, ss2)

        zv = jnp.zeros((L,), jnp.float32)
        hrow0 = hrow3.at[0]
        wbuf0 = wbuf3.at[0]

        def zero_body(i, _):
            for cc in range(H):
                hrow0[i, pl.ds(cc * L, L)] = zv
            wbuf0[i, :] = zv
            return 0

        lax.fori_loop(0, BLK, zero_body, 0)

        off = 0
        rem = rows_per_tile
        while rem > 0:
            sz = min(BLK, rem)
            pltpu.sync_copy(hrow0.at[pl.ds(0, sz)],
                            num_sp.at[pl.ds(r0 + off, sz)])
            pltpu.sync_copy(wbuf0.at[pl.ds(0, sz)],
                            den_sp.at[pl.ds(r0 + off, sz)])
            off += sz
            rem -= sz

        plsc.subcore_barrier()

        def issue_idx(b, slot):
            o = (base_blk + b) * BLK
            pltpu.async_copy(src_hbm.at[pl.ds(o, BLK)], idx6.at[slot, 0], si)
            pltpu.async_copy(dst_hbm.at[pl.ds(o, BLK)], idx6.at[slot, 1], si)

        def wait_idx(slot):
            pltpu.make_async_copy(src_hbm.at[pl.ds(0, BLK)],
                                  idx6.at[slot, 0], si).wait()
            pltpu.make_async_copy(src_hbm.at[pl.ds(0, BLK)],
                                  idx6.at[slot, 1], si).wait()

        def issue_gathers(q, kp, slot):
            pltpu.async_copy(hx_hbm.at[idx6.at[slot, 0]], hb2.at[kp], sg[q])
            pltpu.async_copy(acat_hbm.at[idx6.at[slot, 1]],
                             abuf_d3.at[q], sg[q])

        def drain_gathers(q, kp):
            pltpu.make_async_copy(hx_hbm.at[idx6.at[0, 0]],
                                  hb2.at[kp], sg[q]).wait()
            pltpu.make_async_copy(acat_hbm.at[idx6.at[0, 0]],
                                  abuf_d3.at[q], sg[q]).wait()

        def issue_scatters(q, slot):
            pltpu.async_copy(hrow3.at[q], num_sp.at[idx6.at[slot, 1]],
                             ss[q], add=True)
            pltpu.async_copy(wbuf3.at[q], den_sp.at[idx6.at[slot, 1]],
                             ss[q], add=True)

        def drain_scatters(q):
            pltpu.make_async_copy(hrow3.at[q],
                                  num_sp.at[idx6.at[0, 1]], ss[q]).wait()
            pltpu.make_async_copy(wbuf3.at[q],
                                  den_sp.at[idx6.at[0, 1]], ss[q]).wait()

        pltpu.sync_copy(src_hbm.at[pl.ds(base_blk * BLK, BLK)],
                        idx6.at[0, 0])
        pltpu.sync_copy(dst_hbm.at[pl.ds(base_blk * BLK, BLK)],
                        idx6.at[0, 1])
        issue_idx(1, 1)
        issue_gathers(0, 0, 0)

        def process(b, k6):
            q = k6 % 3
            qn = (q + 1) % 3
            kp = k6 % 2

            @pl.when(b >= 2)
            def _():
                drain_scatters(qn)

            @pl.when(b + 1 < my_nb)
            def _():
                wait_idx((k6 + 1) % 6)

            @pl.when(b + 2 < my_nb)
            def _():
                issue_idx(b + 2, (k6 + 2) % 6)

            @pl.when(b + 1 < my_nb)
            def _():
                issue_gathers(qn, 1 - kp, (k6 + 1) % 6)

            drain_gathers(q, kp)

            hb = hb2.at[kp]
            abuf_d = abuf_d3.at[q]
            hrow = hrow3.at[q]
            wbuf = wbuf3.at[q]

            @plsc.parallel_loop(0, BLK, step=1, unroll=4)
            def edge_body(j):
                v_s = plsc.bitcast(hb[j, pl.ds(HW - 24, L)], jnp.float32)
                v_d = abuf_d[j, :]
                t = v_s + v_d
                w = jnp.exp(jnp.maximum(t, t * NEG_SLOPE))
                wbuf[j, :] = w
                for hh in range(4):
                    v = hb[j, pl.ds(hh * L, L)]
                    va = plsc.bitcast(v << 16, jnp.float32)
                    vb = plsc.bitcast(v & jnp.int32(-65536), jnp.float32)
                    wa = _gather16(w, jnp.full((L,), 8 + hh, jnp.int32))
                    wb = _gather16(w, jnp.full((L,), 12 + hh, jnp.int32))
                    hrow[j, pl.ds(hh * L, L)] = va * wa
                    hrow[j, pl.ds(64 + hh * L, L)] = vb * wb

            issue_scatters(q, k6)

        def block_body(i, _):
            b = i * 6
            for k6 in range(6):
                process(b + k6, k6)
            return 0

        lax.fori_loop(0, my_nb // 6, block_body, 0)

        drain_scatters(1)
        drain_scatters(2)

        plsc.subcore_barrier()

        pltpu.sync_copy(num_sp.at[pl.ds(r0, rows_per_tile)],
                        num_out.at[c, pl.ds(r0, rows_per_tile)])
        pltpu.sync_copy(den_sp.at[pl.ds(r0, rows_per_tile)],
                        den_out.at[c, pl.ds(r0, rows_per_tile)])

    return k


def _bf16_bits(x):
    b = lax.bitcast_convert_type(x, jnp.int32)
    return (b + jnp.int32(0x7FFF) + ((b >> 16) & 1)) >> 16


def _proj_body(x_ref, w_ref, ab_ref, hx_ref, acat_ref):
    n = x_ref.shape[0]
    pad = hx_ref.shape[0] - n
    half = H * C // 2
    hv = jnp.dot(x_ref[...], w_ref[...], preferred_element_type=jnp.float32)
    acat = jnp.dot(hv, ab_ref[...], preferred_element_type=jnp.float32)
    lo = _bf16_bits(hv[:, :half]) & jnp.int32(0xFFFF)
    hi = _bf16_bits(hv[:, half:]) << 16
    hx_ref[pl.ds(0, n), pl.ds(0, half)] = lo | hi
    hx_ref[pl.ds(0, n), pl.ds(half, H)] = lax.bitcast_convert_type(
        acat[:, :H], jnp.int32)
    hx_ref[pl.ds(0, n), pl.ds(half + H, HW - half - H)] = jnp.zeros(
        (n, HW - half - H), jnp.int32)
    hx_ref[pl.ds(n, pad)] = jnp.zeros((pad, HW), jnp.int32)
    acat_ref[pl.ds(0, n)] = acat
    acat_ref[pl.ds(n, pad)] = jnp.zeros((pad, 2 * H), jnp.float32)


def _combine_body(n0_ref, n1_ref, d0_ref, d1_ref, p8_ref, b_ref, o_ref):
    n = o_ref.shape[0]
    dsum = d0_ref[pl.ds(0, n)] + d1_ref[pl.ds(0, n)]
    den = jnp.dot(dsum[:, H:], p8_ref[...],
                  preferred_element_type=jnp.float32) + 1e-16
    o_ref[...] = ((n0_ref[pl.ds(0, n)] + n1_ref[pl.ds(0, n)]) / den
                  + b_ref[...])


def kernel(x, edge_index, W, att_src, att_dst, bias):
    n, f_in = x.shape
    e = edge_index.shape[1]
    f_out = W.shape[1]

    np_rows = -(-(n + 1) // (NS * 8)) * (NS * 8)

    eye = jnp.eye(H, dtype=jnp.float32)
    a_s = (att_src.reshape(H, C)[:, :, None] * eye[:, None, :]).reshape(H * C, H)
    a_d = (att_dst.reshape(H, C)[:, :, None] * eye[:, None, :]).reshape(H * C, H)
    ab = jnp.concatenate([a_s, a_d], axis=1)

    hx, acat = pl.pallas_call(
        _proj_body,
        out_shape=[
            jax.ShapeDtypeStruct((np_rows, HW), jnp.int32),
            jax.ShapeDtypeStruct((np_rows, 2 * H), jnp.float32),
        ],
    )(x, W, ab)

    total = e + n
    nbw = -(-total // (NC * NS * BLK))
    nbw = -(-nbw // 6) * 6
    nbw0 = (nbw * 2 * 4) // 7
    nbw0 = -(-nbw0 // 6) * 6
    nbw1 = 2 * nbw - nbw0
    e_pad = (nbw0 + nbw1) * NS * BLK
    loops = jnp.arange(n, dtype=jnp.int32)
    fill = jnp.full((e_pad - total,), n, jnp.int32)
    src_flat = jnp.concatenate([edge_index[0], loops, fill])
    dst_flat = jnp.concatenate([edge_index[1], loops, fill])

    num_parts, den_parts = _sc_edge_kernel(np_rows, nbw0, nbw1)(
        hx, acat, src_flat, dst_flat)

    p8 = jnp.repeat(jnp.eye(H, dtype=jnp.float32), C, axis=1)
    out = pl.pallas_call(
        _combine_body,
        out_shape=jax.ShapeDtypeStruct((n, f_out), jnp.float32),
    )(num_parts[0], num_parts[1], den_parts[0], den_parts[1], p8,
      bias.reshape(1, f_out))

    return out


# speedup vs baseline: 1.5636x; 1.0332x over previous
import functools

import jax
import jax.numpy as jnp
from jax import lax
from jax.experimental import pallas as pl
from jax.experimental.pallas import tpu as pltpu
from jax.experimental.pallas import tpu_sc as plsc

NC = 2
NS = 16
L = 16
BLK = 56
HW = 80
H = 8
C = 16
NEG_SLOPE = 0.2


def _gather16(v, idx):
    return lax.gather(
        v,
        idx[:, None],
        lax.GatherDimensionNumbers(
            offset_dims=(), collapsed_slice_dims=(0,), start_index_map=(0,)
        ),
        (1,),
        mode=lax.GatherScatterMode.PROMISE_IN_BOUNDS,
    )


def _sc_edge_kernel(np_rows, nbw0, nbw1):
    rows_per_tile = np_rows // NS
    mesh = plsc.VectorSubcoreMesh(core_axis_name="c", subcore_axis_name="s",
                                  num_cores=NC, num_subcores=NS)

    @functools.partial(
        pl.kernel,
        out_type=[
            jax.ShapeDtypeStruct((NC, np_rows, H * C), jnp.float32),
            jax.ShapeDtypeStruct((NC, np_rows, L), jnp.float32),
        ],
        mesh=mesh,
        compiler_params=pltpu.CompilerParams(use_tc_tiling_on_sc=False,
                                             needs_layout_passes=False),
        scratch_types=[
            pltpu.VMEM_SHARED((np_rows, H * C), jnp.float32),
            pltpu.VMEM_SHARED((np_rows, L), jnp.float32),
            pltpu.VMEM((6, 2, BLK), jnp.int32),
            pltpu.VMEM((2, BLK, HW), jnp.int32),
            pltpu.VMEM((3, BLK, L), jnp.float32),
            pltpu.VMEM((3, BLK, H * C), jnp.float32),
            pltpu.VMEM((3, BLK, L), jnp.float32),
            pltpu.SemaphoreType.DMA,
            pltpu.SemaphoreType.DMA,
            pltpu.SemaphoreType.DMA,
            pltpu.SemaphoreType.DMA,
            pltpu.SemaphoreType.DMA,
            pltpu.SemaphoreType.DMA,
            pltpu.SemaphoreType.DMA,
        ],
    )
    def k(hx_hbm, acat_hbm, src_hbm, dst_hbm, num_out, den_out,
          num_sp, den_sp, idx6, hb2, abuf_d3, hrow3, wbuf3,
          sg0, sg1, sg2, ss0, ss1, ss2, si):
        c = lax.axis_index("c")
        s = lax.axis_index("s")
        r0 = s * rows_per_tile
        base_blk = jnp.where(c == 0, s * nbw0, NS * nbw0 + s * nbw1)
        my_nb = jnp.where(c == 0, nbw0, nbw1)
        sg = (sg0, sg1, sg2)
        ss = (ss0, ss1, ss2)

        zv = jnp.zeros((L,), jnp.float32)
        hrow0 = hrow3.at[0]
        wbuf0 = wbuf3.at[0]

        def zero_body(i, _):
            for cc in range(H):
                hrow0[i, pl.ds(cc * L, L)] = zv
            wbuf0[i, :] = zv
            return 0

        lax.fori_loop(0, BLK, zero_body, 0)

        off = 0
        rem = rows_per_tile
        while rem > 0:
            sz = min(BLK, rem)
            pltpu.sync_copy(hrow0.at[pl.ds(0, sz)],
                            num_sp.at[pl.ds(r0 + off, sz)])
            pltpu.sync_copy(wbuf0.at[pl.ds(0, sz)],
                            den_sp.at[pl.ds(r0 + off, sz)])
            off += sz
            rem -= sz

        plsc.subcore_barrier()

        def issue_idx(b, slot):
            o = (base_blk + b) * BLK
            pltpu.async_copy(src_hbm.at[pl.ds(o, BLK)], idx6.at[slot, 0], si)
            pltpu.async_copy(dst_hbm.at[pl.ds(o, BLK)], idx6.at[slot, 1], si)

        def wait_idx(slot):
            pltpu.make_async_copy(src_hbm.at[pl.ds(0, BLK)],
                                  idx6.at[slot, 0], si).wait()
            pltpu.make_async_copy(src_hbm.at[pl.ds(0, BLK)],
                                  idx6.at[slot, 1], si).wait()

        def issue_gathers(q, kp, slot):
            pltpu.async_copy(hx_hbm.at[idx6.at[slot, 0]], hb2.at[kp], sg[q])
            pltpu.async_copy(acat_hbm.at[idx6.at[slot, 1]],
                             abuf_d3.at[q], sg[q])

        def drain_gathers(q, kp):
            pltpu.make_async_copy(hx_hbm.at[idx6.at[0, 0]],
                                  hb2.at[kp], sg[q]).wait()
            pltpu.make_async_copy(acat_hbm.at[idx6.at[0, 0]],
                                  abuf_d3.at[q], sg[q]).wait()

        def issue_scatters(q, slot):
            pltpu.async_copy(hrow3.at[q], num_sp.at[idx6.at[slot, 1]],
                             ss[q], add=True)
            pltpu.async_copy(wbuf3.at[q], den_sp.at[idx6.at[slot, 1]],
                             ss[q], add=True)

        def drain_scatters(q):
            pltpu.make_async_copy(hrow3.at[q],
                                  num_sp.at[idx6.at[0, 1]], ss[q]).wait()
            pltpu.make_async_copy(wbuf3.at[q],
                                  den_sp.at[idx6.at[0, 1]], ss[q]).wait()

        pltpu.sync_copy(src_hbm.at[pl.ds(base_blk * BLK, BLK)],
                        idx6.at[0, 0])
        pltpu.sync_copy(dst_hbm.at[pl.ds(base_blk * BLK, BLK)],
                        idx6.at[0, 1])
        issue_idx(1, 1)
        issue_gathers(0, 0, 0)

        def process(b, k6):
            q = k6 % 3
            qn = (q + 1) % 3
            kp = k6 % 2

            @pl.when(b >= 2)
            def _():
                drain_scatters(qn)

            @pl.when(b + 1 < my_nb)
            def _():
                wait_idx((k6 + 1) % 6)

            @pl.when(b + 2 < my_nb)
            def _():
                issue_idx(b + 2, (k6 + 2) % 6)

            @pl.when(b + 1 < my_nb)
            def _():
                issue_gathers(qn, 1 - kp, (k6 + 1) % 6)

            drain_gathers(q, kp)

            hb = hb2.at[kp]
            abuf_d = abuf_d3.at[q]
            hrow = hrow3.at[q]
            wbuf = wbuf3.at[q]

            @plsc.parallel_loop(0, BLK, step=1, unroll=2)
            def edge_body(j):
                v_s = plsc.bitcast(hb[j, pl.ds(HW - 24, L)], jnp.float32)
                v_d = abuf_d[j, :]
                t = v_s + v_d
                w = jnp.exp(jnp.maximum(t, t * NEG_SLOPE))
                wbuf[j, :] = w
                for hh in range(4):
                    v = hb[j, pl.ds(hh * L, L)]
                    va = plsc.bitcast(v << 16, jnp.float32)
                    vb = plsc.bitcast(v & jnp.int32(-65536), jnp.float32)
                    wa = _gather16(w, jnp.full((L,), 8 + hh, jnp.int32))
                    wb = _gather16(w, jnp.full((L,), 12 + hh, jnp.int32))
                    hrow[j, pl.ds(hh * L, L)] = va * wa
                    hrow[j, pl.ds(64 + hh * L, L)] = vb * wb

            issue_scatters(q, k6)

        def block_body(i, _):
            b = i * 6
            for k6 in range(6):
                process(b + k6, k6)
            return 0

        lax.fori_loop(0, my_nb // 6, block_body, 0)

        drain_scatters(1)
        drain_scatters(2)

        plsc.subcore_barrier()

        pltpu.sync_copy(num_sp.at[pl.ds(r0, rows_per_tile)],
                        num_out.at[c, pl.ds(r0, rows_per_tile)])
        pltpu.sync_copy(den_sp.at[pl.ds(r0, rows_per_tile)],
                        den_out.at[c, pl.ds(r0, rows_per_tile)])

    return k


def _bf16_bits(x):
    b = lax.bitcast_convert_type(x, jnp.int32)
    return (b + jnp.int32(0x7FFF) + ((b >> 16) & 1)) >> 16


def _proj_body(x_ref, w_ref, ab_ref, hx_ref, acat_ref):
    n = x_ref.shape[0]
    pad = hx_ref.shape[0] - n
    half = H * C // 2
    hv = jnp.dot(x_ref[...], w_ref[...], preferred_element_type=jnp.float32)
    acat = jnp.dot(hv, ab_ref[...], preferred_element_type=jnp.float32)
    lo = _bf16_bits(hv[:, :half]) & jnp.int32(0xFFFF)
    hi = _bf16_bits(hv[:, half:]) << 16
    hx_ref[pl.ds(0, n), pl.ds(0, half)] = lo | hi
    hx_ref[pl.ds(0, n), pl.ds(half, H)] = lax.bitcast_convert_type(
        acat[:, :H], jnp.int32)
    hx_ref[pl.ds(0, n), pl.ds(half + H, HW - half - H)] = jnp.zeros(
        (n, HW - half - H), jnp.int32)
    hx_ref[pl.ds(n, pad)] = jnp.zeros((pad, HW), jnp.int32)
    acat_ref[pl.ds(0, n)] = acat
    acat_ref[pl.ds(n, pad)] = jnp.zeros((pad, 2 * H), jnp.float32)


def _combine_body(n0_ref, n1_ref, d0_ref, d1_ref, p8_ref, b_ref, o_ref):
    n = o_ref.shape[0]
    dsum = d0_ref[pl.ds(0, n)] + d1_ref[pl.ds(0, n)]
    den = jnp.dot(dsum[:, H:], p8_ref[...],
                  preferred_element_type=jnp.float32) + 1e-16
    o_ref[...] = ((n0_ref[pl.ds(0, n)] + n1_ref[pl.ds(0, n)]) / den
                  + b_ref[...])


def kernel(x, edge_index, W, att_src, att_dst, bias):
    n, f_in = x.shape
    e = edge_index.shape[1]
    f_out = W.shape[1]

    np_rows = -(-(n + 1) // (NS * 8)) * (NS * 8)

    eye = jnp.eye(H, dtype=jnp.float32)
    a_s = (att_src.reshape(H, C)[:, :, None] * eye[:, None, :]).reshape(H * C, H)
    a_d = (att_dst.reshape(H, C)[:, :, None] * eye[:, None, :]).reshape(H * C, H)
    ab = jnp.concatenate([a_s, a_d], axis=1)

    hx, acat = pl.pallas_call(
        _proj_body,
        out_shape=[
            jax.ShapeDtypeStruct((np_rows, HW), jnp.int32),
            jax.ShapeDtypeStruct((np_rows, 2 * H), jnp.float32),
        ],
    )(x, W, ab)

    total = e + n
    nbw = -(-total // (NC * NS * BLK))
    nbw = -(-nbw // 6) * 6
    nbw0 = (nbw * 2 * 61) // 100
    nbw0 = -(-nbw0 // 6) * 6
    nbw1 = 2 * nbw - nbw0
    e_pad = (nbw0 + nbw1) * NS * BLK
    loops = jnp.arange(n, dtype=jnp.int32)
    fill = jnp.full((e_pad - total,), n, jnp.int32)
    src_flat = jnp.concatenate([edge_index[0], loops, fill])
    dst_flat = jnp.concatenate([edge_index[1], loops, fill])

    num_parts, den_parts = _sc_edge_kernel(np_rows, nbw0, nbw1)(
        hx, acat, src_flat, dst_flat)

    p8 = jnp.repeat(jnp.eye(H, dtype=jnp.float32), C, axis=1)
    out = pl.pallas_call(
        _combine_body,
        out_shape=jax.ShapeDtypeStruct((n, f_out), jnp.float32),
    )(num_parts[0], num_parts[1], den_parts[0], den_parts[1], p8,
      bias.reshape(1, f_out))

    return out
